# baseline (speedup 1.0000x reference)
import jax
import jax.numpy as jnp
from jax import lax
from jax.experimental import pallas as pl
from jax.experimental.pallas import tpu as pltpu
from jax.experimental.pallas import tpu_sc as plsc

N = 10000
E = 320000
U = 128

NC = 2
NS = 16
LANES = 16

H = U // NC
CHUNK = 128
NCHUNKS = E // CHUNK
TRIPS = (NCHUNKS + NS - 1) // NS
ROWS_PER_TILE = N // NS
ZROWS = 125


def _sigmoid(v):
    return 1.0 / (1.0 + jnp.exp(-v))


def _sc_body(src_hbm, dst_hbm, w0_hbm, x2_hbm, x3_hbm, x4_hbm,
             seg_hbm, cnt_hbm, g34_hbm,
             idx_src, idx_dst, w0c, x2r, g3, g4, ones16, zbuf, zcnt,
             seg_acc, cnt_acc,
             sem_i, sem_g2, sem_g34, sem_s, sem_c, sem_o):
    cid = lax.axis_index("c")
    sid = lax.axis_index("s")

    zeros16 = jnp.zeros((LANES,), jnp.float32)
    ones = jnp.ones((LANES,), jnp.float32)

    def init_ones(i, _):
        ones16[i, :] = ones
        return 0
    lax.fori_loop(0, CHUNK, init_ones, 0)

    def init_zbuf(i, _):
        for j in range(H // LANES):
            zbuf[i, pl.ds(j * LANES, LANES)] = zeros16
        zcnt[i, :] = zeros16
        return 0
    lax.fori_loop(0, ZROWS, init_zbuf, 0)

    for kk in range(ROWS_PER_TILE // ZROWS):
        off = sid * ROWS_PER_TILE + kk * ZROWS
        pltpu.sync_copy(zbuf, seg_acc.at[pl.ds(off, ZROWS)])
        pltpu.sync_copy(zcnt, cnt_acc.at[pl.ds(off, ZROWS)])
    plsc.subcore_barrier()

    def d_src(q, k):
        return pltpu.make_async_copy(
            src_hbm.at[pl.ds(k * CHUNK, CHUNK)], idx_src.at[q], sem_i.at[q])

    def d_dst(q, k):
        return pltpu.make_async_copy(
            dst_hbm.at[pl.ds(k * CHUNK, CHUNK)], idx_dst.at[q], sem_i.at[q])

    def d_w0(q, k):
        return pltpu.make_async_copy(
            w0_hbm.at[pl.ds(k * CHUNK, CHUNK), pl.ds(cid * H, H)],
            w0c.at[pl.ds(q * CHUNK, CHUNK)], sem_i.at[q])

    def d_x2(q):
        return pltpu.make_async_copy(
            x2_hbm.at[cid].at[idx_dst.at[q]],
            x2r.at[pl.ds(q * CHUNK, CHUNK)], sem_g2.at[q])

    def d_x3(q):
        return pltpu.make_async_copy(
            x3_hbm.at[cid].at[idx_src.at[q]],
            g3.at[pl.ds(q * CHUNK, CHUNK)], sem_g34.at[q])

    def d_x4(q):
        return pltpu.make_async_copy(
            x4_hbm.at[cid].at[idx_dst.at[q]],
            g4.at[pl.ds(q * CHUNK, CHUNK)], sem_g34.at[q])

    def d_seg(q):
        return pltpu.make_async_copy(
            x2r.at[pl.ds(q * CHUNK, CHUNK)],
            seg_acc.at[idx_src.at[q]], sem_s.at[q])

    def d_cnt(q):
        return pltpu.make_async_copy(
            ones16, cnt_acc.at[idx_src.at[q]], sem_c.at[q])

    def d_g34(q, k):
        return pltpu.make_async_copy(
            g3.at[pl.ds(q * CHUNK, CHUNK)],
            g34_hbm.at[cid].at[pl.ds(k * CHUNK, CHUNK)], sem_o.at[q])

    d_src(0, sid).start()
    d_dst(0, sid).start()
    d_w0(0, sid).start()

    def trip(t, _):
        p = lax.rem(t, 2)
        np_ = 1 - p
        k = sid + t * NS

        @pl.when(k < NCHUNKS)
        def _():
            @pl.when(t >= 1)
            def _():
                d_seg(np_).wait()

                @pl.when(cid == 0)
                def _():
                    d_cnt(np_).wait()
                d_g34(np_, k - NS).wait()

            d_src(p, k).wait()
            d_dst(p, k).wait()
            d_w0(p, k).wait()

            d_x2(p).start()
            d_x3(p).start()
            d_x4(p).start()

            @pl.when(k + NS < NCHUNKS)
            def _():
                d_src(np_, k + NS).start()
                d_dst(np_, k + NS).start()
                d_w0(np_, k + NS).start()

            d_x2(p).wait()
            row0 = p * CHUNK

            def msg_row(i, _):
                for j in range(H // LANES):
                    sl = pl.ds(j * LANES, LANES)
                    x2r[row0 + i, sl] = x2r[row0 + i, sl] * _sigmoid(
                        w0c[row0 + i, sl])
                return 0
            lax.fori_loop(0, CHUNK, msg_row, 0)

            d_seg(p).start(add=True)

            @pl.when(cid == 0)
            def _():
                d_cnt(p).start(add=True)

            d_x3(p).wait()
            d_x4(p).wait()

            def add_row(i, _):
                for j in range(H // LANES):
                    sl = pl.ds(j * LANES, LANES)
                    g3[row0 + i, sl] = g3[row0 + i, sl] + g4[row0 + i, sl]
                return 0
            lax.fori_loop(0, CHUNK, add_row, 0)
            d_g34(p, k).start()
        return 0

    lax.fori_loop(0, TRIPS, trip, 0)

    last_t_full = TRIPS - 1
    for td in (TRIPS - 1, TRIPS - 2):
        ran_td = sid + td * NS < NCHUNKS
        ran_next = sid + (td + 1) * NS < NCHUNKS

        @pl.when(ran_td & jnp.logical_not(ran_next))
        def _():
            q = td % 2
            d_seg(q).wait()

            @pl.when(cid == 0)
            def _():
                d_cnt(q).wait()
            d_g34(q, sid + td * NS).wait()

    plsc.subcore_barrier()

    @pl.when(sid == 0)
    def _():
        pltpu.sync_copy(seg_acc, seg_hbm.at[cid])

        @pl.when(cid == 0)
        def _():
            pltpu.sync_copy(cnt_acc, cnt_hbm)


def _sc_call(src, dst, w0, x2s, x3s, x4s):
    mesh = plsc.VectorSubcoreMesh(core_axis_name="c", subcore_axis_name="s")
    f = pl.kernel(
        _sc_body,
        out_type=(
            jax.ShapeDtypeStruct((NC, N, H), jnp.float32),
            jax.ShapeDtypeStruct((N, LANES), jnp.float32),
            jax.ShapeDtypeStruct((NC, E, H), jnp.float32),
        ),
        mesh=mesh,
        compiler_params=pltpu.CompilerParams(use_tc_tiling_on_sc=False),
        scratch_types=[
            pltpu.VMEM((2, CHUNK), jnp.int32),
            pltpu.VMEM((2, CHUNK), jnp.int32),
            pltpu.VMEM((2 * CHUNK, H), jnp.float32),
            pltpu.VMEM((2 * CHUNK, H), jnp.float32),
            pltpu.VMEM((2 * CHUNK, H), jnp.float32),
            pltpu.VMEM((2 * CHUNK, H), jnp.float32),
            pltpu.VMEM((CHUNK, LANES), jnp.float32),
            pltpu.VMEM((ZROWS, H), jnp.float32),
            pltpu.VMEM((ZROWS, LANES), jnp.float32),
            pltpu.VMEM_SHARED((N, H), jnp.float32),
            pltpu.VMEM_SHARED((N, LANES), jnp.float32),
            pltpu.SemaphoreType.DMA((2,)),
            pltpu.SemaphoreType.DMA((2,)),
            pltpu.SemaphoreType.DMA((2,)),
            pltpu.SemaphoreType.DMA((2,)),
            pltpu.SemaphoreType.DMA((2,)),
            pltpu.SemaphoreType.DMA((2,)),
        ],
    )
    return f(src, dst, w0, x2s, x3s, x4s)


def _node_mm_body(x_ref, wt_ref, b_ref, o_ref):
    o_ref[...] = (
        jnp.dot(x_ref[...], wt_ref[...], preferred_element_type=jnp.float32)
        + b_ref[...]
    )


def _node_out_body(x0_ref, x1_ref, segp_ref, cnt_ref, g_ref, b_ref, o_ref):
    seg = jnp.concatenate([segp_ref[0], segp_ref[1]], axis=1)
    cnt = jnp.sum(cnt_ref[...], axis=1, keepdims=True) * (1.0 / LANES)
    pooled = seg / jnp.maximum(cnt, 1.0)
    h = x1_ref[...] + pooled
    mu = jnp.mean(h, axis=0, keepdims=True)
    d = h - mu
    var = jnp.mean(d * d, axis=0, keepdims=True)
    z = g_ref[...] * d * lax.rsqrt(var + 1e-5) + b_ref[...]
    o_ref[...] = x0_ref[...] + z * _sigmoid(z)


EB = 3200


def _edge_pre_body(w0_ref, g34_ref, wet_ref, be_ref, ep_ref, s_ref, q_ref):
    g34 = jnp.concatenate([g34_ref[0], g34_ref[1]], axis=1)
    ep = (
        jnp.dot(w0_ref[...], wet_ref[...], preferred_element_type=jnp.float32)
        + be_ref[...]
        + g34
    )
    ep_ref[...] = ep
    bs = jnp.sum(ep, axis=0, keepdims=True)
    bq = jnp.sum(ep * ep, axis=0, keepdims=True)

    @pl.when(pl.program_id(0) == 0)
    def _():
        s_ref[...] = bs
        q_ref[...] = bq

    @pl.when(pl.program_id(0) > 0)
    def _():
        s_ref[...] += bs
        q_ref[...] += bq


def _edge_out_body(w0_ref, ep_ref, s_ref, q_ref, g_ref, b_ref, o_ref):
    inv_e = 1.0 / E
    mu = s_ref[...] * inv_e
    var = q_ref[...] * inv_e - mu * mu
    z = g_ref[...] * (ep_ref[...] - mu) * lax.rsqrt(var + 1e-5) + b_ref[...]
    o_ref[...] = w0_ref[...] + z * _sigmoid(z)


def kernel(x, edge_index, edge_attr, w_v1, b_v1, w_v2, b_v2, w_v3, b_v3,
           w_v4, b_v4, w_e, b_e, bn_v_gamma, bn_v_beta, bn_e_gamma, bn_e_beta):
    src = edge_index[0]
    dst = edge_index[1]

    wt = jnp.concatenate([w_v1.T, w_v2.T, w_v3.T, w_v4.T], axis=1)
    bc = jnp.concatenate([b_v1, b_v2, b_v3, b_v4]).reshape(1, 4 * U)
    x1234 = pl.pallas_call(
        _node_mm_body,
        out_shape=jax.ShapeDtypeStruct((N, 4 * U), jnp.float32),
    )(x, wt, bc)
    x1 = x1234[:, :U]

    def _halves(a):
        return jnp.stack([a[:, :H], a[:, H:]])

    x2s = _halves(x1234[:, U:2 * U])
    x3s = _halves(x1234[:, 2 * U:3 * U])
    x4s = _halves(x1234[:, 3 * U:])

    seg_parts, cnt16, g34s = _sc_call(src, dst, edge_attr, x2s, x3s, x4s)

    x_out = pl.pallas_call(
        _node_out_body,
        out_shape=jax.ShapeDtypeStruct((N, U), jnp.float32),
    )(x, x1, seg_parts, cnt16,
      bn_v_gamma.reshape(1, U), bn_v_beta.reshape(1, U))

    grid = E // EB
    e_pre, ssum, ssq = pl.pallas_call(
        _edge_pre_body,
        grid=(grid,),
        in_specs=[
            pl.BlockSpec((EB, U), lambda i: (i, 0)),
            pl.BlockSpec((NC, EB, H), lambda i: (0, i, 0)),
            pl.BlockSpec((U, U), lambda i: (0, 0)),
            pl.BlockSpec((1, U), lambda i: (0, 0)),
        ],
        out_specs=[
            pl.BlockSpec((EB, U), lambda i: (i, 0)),
            pl.BlockSpec((1, U), lambda i: (0, 0)),
            pl.BlockSpec((1, U), lambda i: (0, 0)),
        ],
        out_shape=[
            jax.ShapeDtypeStruct((E, U), jnp.float32),
            jax.ShapeDtypeStruct((1, U), jnp.float32),
            jax.ShapeDtypeStruct((1, U), jnp.float32),
        ],
    )(edge_attr, g34s, w_e.T, b_e.reshape(1, U))

    w_out = pl.pallas_call(
        _edge_out_body,
        grid=(grid,),
        in_specs=[
            pl.BlockSpec((EB, U), lambda i: (i, 0)),
            pl.BlockSpec((EB, U), lambda i: (i, 0)),
            pl.BlockSpec((1, U), lambda i: (0, 0)),
            pl.BlockSpec((1, U), lambda i: (0, 0)),
            pl.BlockSpec((1, U), lambda i: (0, 0)),
            pl.BlockSpec((1, U), lambda i: (0, 0)),
        ],
        out_specs=pl.BlockSpec((EB, U), lambda i: (i, 0)),
        out_shape=jax.ShapeDtypeStruct((E, U), jnp.float32),
    )(edge_attr, e_pre, ssum, ssq,
      bn_e_gamma.reshape(1, U), bn_e_beta.reshape(1, U))

    return (x_out, w_out)


# speedup vs baseline: 2.2234x; 2.2234x over previous
import jax
import jax.numpy as jnp
from jax import lax
from jax.experimental import pallas as pl
from jax.experimental.pallas import tpu as pltpu
from jax.experimental.pallas import tpu_sc as plsc

N = 10000
E = 320000
U = 128

NC = 2
NS = 16
LANES = 16

H = U // NC
CHUNK = 128
NCHUNKS = E // CHUNK
TRIPS = (NCHUNKS + NS - 1) // NS
ROWS_PER_TILE = N // NS
ZROWS = 125


def _sigmoid(v):
    return 1.0 / (1.0 + jnp.exp(-v))


def _sc_body(src_hbm, dst_hbm, w0_hbm, x2_hbm, x3_hbm, x4_hbm,
             seg_hbm, cnt_hbm, g34_hbm,
             idx_src, idx_dst, w0c, x2r, g3, g4, ones16, zbuf, zcnt,
             seg_acc, cnt_acc,
             sem_i, sem_g2, sem_g34, sem_s, sem_c, sem_o):
    cid = lax.axis_index("c")
    sid = lax.axis_index("s")

    zeros16 = jnp.zeros((LANES,), jnp.float32)
    ones = jnp.ones((LANES,), jnp.float32)

    def init_ones(i, _):
        ones16[i, :] = ones
        return 0
    lax.fori_loop(0, CHUNK, init_ones, 0)

    def init_zbuf(i, _):
        for j in range(H // LANES):
            zbuf[i, pl.ds(j * LANES, LANES)] = zeros16
        zcnt[i, :] = zeros16
        return 0
    lax.fori_loop(0, ZROWS, init_zbuf, 0)

    for kk in range(ROWS_PER_TILE // ZROWS):
        off = sid * ROWS_PER_TILE + kk * ZROWS
        pltpu.sync_copy(zbuf, seg_acc.at[pl.ds(off, ZROWS)])
        pltpu.sync_copy(zcnt, cnt_acc.at[pl.ds(off, ZROWS)])
    plsc.subcore_barrier()

    def d_src(q, k):
        return pltpu.make_async_copy(
            src_hbm.at[pl.ds(k * CHUNK, CHUNK)], idx_src.at[q], sem_i.at[q])

    def d_dst(q, k):
        return pltpu.make_async_copy(
            dst_hbm.at[pl.ds(k * CHUNK, CHUNK)], idx_dst.at[q], sem_i.at[q])

    def d_w0(q, k):
        return pltpu.make_async_copy(
            w0_hbm.at[pl.ds(k * CHUNK, CHUNK), pl.ds(cid * H, H)],
            w0c.at[pl.ds(q * CHUNK, CHUNK)], sem_i.at[q])

    def d_x2(q):
        return pltpu.make_async_copy(
            x2_hbm.at[cid].at[idx_dst.at[q]],
            x2r.at[pl.ds(q * CHUNK, CHUNK)], sem_g2.at[q])

    def d_x3(q):
        return pltpu.make_async_copy(
            x3_hbm.at[cid].at[idx_src.at[q]],
            g3.at[pl.ds(q * CHUNK, CHUNK)], sem_g34.at[q])

    def d_x4(q):
        return pltpu.make_async_copy(
            x4_hbm.at[cid].at[idx_dst.at[q]],
            g4.at[pl.ds(q * CHUNK, CHUNK)], sem_g34.at[q])

    def d_seg(q):
        return pltpu.make_async_copy(
            x2r.at[pl.ds(q * CHUNK, CHUNK)],
            seg_acc.at[idx_src.at[q]], sem_s.at[q])

    def d_cnt(q):
        return pltpu.make_async_copy(
            ones16, cnt_acc.at[idx_src.at[q]], sem_c.at[q])

    def d_g34(q, k):
        return pltpu.make_async_copy(
            g3.at[pl.ds(q * CHUNK, CHUNK)],
            g34_hbm.at[cid].at[pl.ds(k * CHUNK, CHUNK)], sem_o.at[q])

    def trip(t, _):
        k = sid + t * NS

        @pl.when(k < NCHUNKS)
        def _():
            d_src(0, k).start()
            d_dst(0, k).start()
            d_src(0, k).wait()
            d_dst(0, k).wait()

            d_w0(0, k).start()
            d_x2(0).start()
            d_x3(0).start()
            d_x4(0).start()

            d_w0(0, k).wait()
            d_x2(0).wait()

            def msg_row(i, _):
                for j in range(H // LANES):
                    sl = pl.ds(j * LANES, LANES)
                    x2r[i, sl] = x2r[i, sl] * _sigmoid(w0c[i, sl])
                return 0
            lax.fori_loop(0, CHUNK, msg_row, 0)

            d_seg(0).start(add=True)

            @pl.when(cid == 0)
            def _():
                d_cnt(0).start(add=True)

            d_x3(0).wait()
            d_x4(0).wait()

            def add_row(i, _):
                for j in range(H // LANES):
                    sl = pl.ds(j * LANES, LANES)
                    g3[i, sl] = g3[i, sl] + g4[i, sl]
                return 0
            lax.fori_loop(0, CHUNK, add_row, 0)
            d_g34(0, k).start()

            d_seg(0).wait()

            @pl.when(cid == 0)
            def _():
                d_cnt(0).wait()
            d_g34(0, k).wait()
        return 0

    lax.fori_loop(0, TRIPS, trip, 0)
    plsc.subcore_barrier()

    @pl.when(sid == 0)
    def _():
        pltpu.sync_copy(seg_acc, seg_hbm.at[cid])

        @pl.when(cid == 0)
        def _():
            pltpu.sync_copy(cnt_acc, cnt_hbm)


def _sc_call(src, dst, w0, x2s, x3s, x4s):
    mesh = plsc.VectorSubcoreMesh(core_axis_name="c", subcore_axis_name="s")
    f = pl.kernel(
        _sc_body,
        out_type=(
            jax.ShapeDtypeStruct((NC, N, H), jnp.float32),
            jax.ShapeDtypeStruct((N, LANES), jnp.float32),
            jax.ShapeDtypeStruct((NC, E, H), jnp.float32),
        ),
        mesh=mesh,
        compiler_params=pltpu.CompilerParams(use_tc_tiling_on_sc=False),
        scratch_types=[
            pltpu.VMEM((2, CHUNK), jnp.int32),
            pltpu.VMEM((2, CHUNK), jnp.int32),
            pltpu.VMEM((2 * CHUNK, H), jnp.float32),
            pltpu.VMEM((2 * CHUNK, H), jnp.float32),
            pltpu.VMEM((2 * CHUNK, H), jnp.float32),
            pltpu.VMEM((2 * CHUNK, H), jnp.float32),
            pltpu.VMEM((CHUNK, LANES), jnp.float32),
            pltpu.VMEM((ZROWS, H), jnp.float32),
            pltpu.VMEM((ZROWS, LANES), jnp.float32),
            pltpu.VMEM_SHARED((N, H), jnp.float32),
            pltpu.VMEM_SHARED((N, LANES), jnp.float32),
            pltpu.SemaphoreType.DMA((2,)),
            pltpu.SemaphoreType.DMA((2,)),
            pltpu.SemaphoreType.DMA((2,)),
            pltpu.SemaphoreType.DMA((2,)),
            pltpu.SemaphoreType.DMA((2,)),
            pltpu.SemaphoreType.DMA((2,)),
        ],
    )
    return f(src, dst, w0, x2s, x3s, x4s)


def _node_mm_body(x_ref, wt_ref, b_ref, o_ref):
    o_ref[...] = (
        jnp.dot(x_ref[...], wt_ref[...], preferred_element_type=jnp.float32)
        + b_ref[...]
    )


def _node_out_body(x0_ref, x1_ref, segp_ref, cnt_ref, g_ref, b_ref, o_ref):
    seg = jnp.concatenate([segp_ref[0], segp_ref[1]], axis=1)
    cnt = jnp.sum(cnt_ref[...], axis=1, keepdims=True) * (1.0 / LANES)
    pooled = seg / jnp.maximum(cnt, 1.0)
    h = x1_ref[...] + pooled
    mu = jnp.mean(h, axis=0, keepdims=True)
    d = h - mu
    var = jnp.mean(d * d, axis=0, keepdims=True)
    z = g_ref[...] * d * lax.rsqrt(var + 1e-5) + b_ref[...]
    o_ref[...] = x0_ref[...] + z * _sigmoid(z)


EB = 3200


def _edge_pre_body(w0_ref, g34_ref, wet_ref, be_ref, ep_ref, s_ref, q_ref):
    g34 = jnp.concatenate([g34_ref[0], g34_ref[1]], axis=1)
    ep = (
        jnp.dot(w0_ref[...], wet_ref[...], preferred_element_type=jnp.float32)
        + be_ref[...]
        + g34
    )
    ep_ref[...] = ep
    bs = jnp.sum(ep, axis=0, keepdims=True)
    bq = jnp.sum(ep * ep, axis=0, keepdims=True)

    @pl.when(pl.program_id(0) == 0)
    def _():
        s_ref[...] = bs
        q_ref[...] = bq

    @pl.when(pl.program_id(0) > 0)
    def _():
        s_ref[...] += bs
        q_ref[...] += bq


def _edge_out_body(w0_ref, ep_ref, s_ref, q_ref, g_ref, b_ref, o_ref):
    inv_e = 1.0 / E
    mu = s_ref[...] * inv_e
    var = q_ref[...] * inv_e - mu * mu
    z = g_ref[...] * (ep_ref[...] - mu) * lax.rsqrt(var + 1e-5) + b_ref[...]
    o_ref[...] = w0_ref[...] + z * _sigmoid(z)


def kernel(x, edge_index, edge_attr, w_v1, b_v1, w_v2, b_v2, w_v3, b_v3,
           w_v4, b_v4, w_e, b_e, bn_v_gamma, bn_v_beta, bn_e_gamma, bn_e_beta):
    src = edge_index[0]
    dst = edge_index[1]

    wt = jnp.concatenate([w_v1.T, w_v2.T, w_v3.T, w_v4.T], axis=1)
    bc = jnp.concatenate([b_v1, b_v2, b_v3, b_v4]).reshape(1, 4 * U)
    x1234 = pl.pallas_call(
        _node_mm_body,
        out_shape=jax.ShapeDtypeStruct((N, 4 * U), jnp.float32),
    )(x, wt, bc)
    x1 = x1234[:, :U]

    def _halves(a):
        return jnp.stack([a[:, :H], a[:, H:]])

    x2s = _halves(x1234[:, U:2 * U])
    x3s = _halves(x1234[:, 2 * U:3 * U])
    x4s = _halves(x1234[:, 3 * U:])

    seg_parts, cnt16, g34s = _sc_call(src, dst, edge_attr, x2s, x3s, x4s)

    x_out = pl.pallas_call(
        _node_out_body,
        out_shape=jax.ShapeDtypeStruct((N, U), jnp.float32),
    )(x, x1, seg_parts, cnt16,
      bn_v_gamma.reshape(1, U), bn_v_beta.reshape(1, U))

    grid = E // EB
    e_pre, ssum, ssq = pl.pallas_call(
        _edge_pre_body,
        grid=(grid,),
        in_specs=[
            pl.BlockSpec((EB, U), lambda i: (i, 0)),
            pl.BlockSpec((NC, EB, H), lambda i: (0, i, 0)),
            pl.BlockSpec((U, U), lambda i: (0, 0)),
            pl.BlockSpec((1, U), lambda i: (0, 0)),
        ],
        out_specs=[
            pl.BlockSpec((EB, U), lambda i: (i, 0)),
            pl.BlockSpec((1, U), lambda i: (0, 0)),
            pl.BlockSpec((1, U), lambda i: (0, 0)),
        ],
        out_shape=[
            jax.ShapeDtypeStruct((E, U), jnp.float32),
            jax.ShapeDtypeStruct((1, U), jnp.float32),
            jax.ShapeDtypeStruct((1, U), jnp.float32),
        ],
    )(edge_attr, g34s, w_e.T, b_e.reshape(1, U))

    w_out = pl.pallas_call(
        _edge_out_body,
        grid=(grid,),
        in_specs=[
            pl.BlockSpec((EB, U), lambda i: (i, 0)),
            pl.BlockSpec((EB, U), lambda i: (i, 0)),
            pl.BlockSpec((1, U), lambda i: (0, 0)),
            pl.BlockSpec((1, U), lambda i: (0, 0)),
            pl.BlockSpec((1, U), lambda i: (0, 0)),
            pl.BlockSpec((1, U), lambda i: (0, 0)),
        ],
        out_specs=pl.BlockSpec((EB, U), lambda i: (i, 0)),
        out_shape=jax.ShapeDtypeStruct((E, U), jnp.float32),
    )(edge_attr, e_pre, ssum, ssq,
      bn_e_gamma.reshape(1, U), bn_e_beta.reshape(1, U))

    return (x_out, w_out)


# speedup vs baseline: 2.8152x; 1.2662x over previous
import jax
import jax.numpy as jnp
from jax import lax
from jax.experimental import pallas as pl
from jax.experimental.pallas import tpu as pltpu
from jax.experimental.pallas import tpu_sc as plsc

N = 10000
E = 320000
U = 128

NC = 2
NS = 16
LANES = 16

H = U // NC
CHUNK = 128
NCHUNKS = E // CHUNK
TRIPS = (NCHUNKS + NS - 1) // NS
ROWS_PER_TILE = N // NS
ZROWS = 125


def _sigmoid(v):
    return 1.0 / (1.0 + jnp.exp(-v))


def _sc_body(src_hbm, dst_hbm, w0_hbm, x2_hbm, x3_hbm, x4_hbm,
             seg_hbm, cnt_hbm, g34_hbm,
             idx_src, idx_dst, w0c, x2r, g3, g4, ones16, zbuf, zcnt,
             seg_acc, cnt_acc,
             sem_i, sem_g2, sem_g34, sem_s, sem_c, sem_o):
    cid = lax.axis_index("c")
    sid = lax.axis_index("s")

    zeros16 = jnp.zeros((LANES,), jnp.float32)
    ones = jnp.ones((LANES,), jnp.float32)

    def init_ones(i, _):
        ones16[i, :] = ones
        return 0
    lax.fori_loop(0, CHUNK, init_ones, 0)

    def init_zbuf(i, _):
        for j in range(H // LANES):
            zbuf[i, pl.ds(j * LANES, LANES)] = zeros16
        zcnt[i, :] = zeros16
        return 0
    lax.fori_loop(0, ZROWS, init_zbuf, 0)

    for kk in range(ROWS_PER_TILE // ZROWS):
        off = sid * ROWS_PER_TILE + kk * ZROWS
        pltpu.sync_copy(zbuf, seg_acc.at[pl.ds(off, ZROWS)])
        pltpu.sync_copy(zcnt, cnt_acc.at[pl.ds(off, ZROWS)])
    plsc.subcore_barrier()

    def d_src(q, k):
        return pltpu.make_async_copy(
            src_hbm.at[pl.ds(k * CHUNK, CHUNK)], idx_src.at[q], sem_i.at[q])

    def d_dst(q, k):
        return pltpu.make_async_copy(
            dst_hbm.at[pl.ds(k * CHUNK, CHUNK)], idx_dst.at[q], sem_i.at[q])

    def d_w0(q, k):
        return pltpu.make_async_copy(
            w0_hbm.at[pl.ds(k * CHUNK, CHUNK), pl.ds(cid * H, H)],
            w0c.at[pl.ds(q * CHUNK, CHUNK)], sem_i.at[q])

    def d_x2(q):
        return pltpu.make_async_copy(
            x2_hbm.at[cid].at[idx_dst.at[q]],
            x2r.at[pl.ds(q * CHUNK, CHUNK)], sem_g2.at[q])

    def d_x3(q):
        return pltpu.make_async_copy(
            x3_hbm.at[cid].at[idx_src.at[q]],
            g3.at[pl.ds(q * CHUNK, CHUNK)], sem_g34.at[q])

    def d_x4(q):
        return pltpu.make_async_copy(
            x4_hbm.at[cid].at[idx_dst.at[q]],
            g4.at[pl.ds(q * CHUNK, CHUNK)], sem_g34.at[q])

    def d_seg(q):
        return pltpu.make_async_copy(
            x2r.at[pl.ds(q * CHUNK, CHUNK)],
            seg_acc.at[idx_src.at[q]], sem_s.at[q])

    def d_cnt(q):
        return pltpu.make_async_copy(
            ones16, cnt_acc.at[idx_src.at[q]], sem_c.at[q])

    def d_g34(q, k):
        return pltpu.make_async_copy(
            g3.at[pl.ds(q * CHUNK, CHUNK)],
            g34_hbm.at[pl.ds(k * CHUNK, CHUNK), pl.ds(cid * H, H)],
            sem_o.at[q])

    def trip(t, _):
        k = sid + t * NS

        @pl.when(k < NCHUNKS)
        def _():
            d_src(0, k).start()
            d_dst(0, k).start()
            d_src(0, k).wait()
            d_dst(0, k).wait()

            d_w0(0, k).start()
            d_x2(0).start()
            d_x3(0).start()
            d_x4(0).start()

            d_w0(0, k).wait()
            d_x2(0).wait()

            def msg_row(i, _):
                for j in range(H // LANES):
                    sl = pl.ds(j * LANES, LANES)
                    x2r[i, sl] = x2r[i, sl] * _sigmoid(w0c[i, sl])
                return 0
            lax.fori_loop(0, CHUNK, msg_row, 0)

            d_seg(0).start(add=True)

            @pl.when(cid == 0)
            def _():
                d_cnt(0).start(add=True)

            d_x3(0).wait()
            d_x4(0).wait()

            def add_row(i, _):
                for j in range(H // LANES):
                    sl = pl.ds(j * LANES, LANES)
                    g3[i, sl] = g3[i, sl] + g4[i, sl]
                return 0
            lax.fori_loop(0, CHUNK, add_row, 0)
            d_g34(0, k).start()

            d_seg(0).wait()

            @pl.when(cid == 0)
            def _():
                d_cnt(0).wait()
            d_g34(0, k).wait()
        return 0

    lax.fori_loop(0, TRIPS, trip, 0)
    plsc.subcore_barrier()

    @pl.when(sid == 0)
    def _():
        pltpu.sync_copy(seg_acc, seg_hbm.at[cid])

        @pl.when(cid == 0)
        def _():
            pltpu.sync_copy(cnt_acc, cnt_hbm)


def _sc_call(src, dst, w0, x2s, x3s, x4s):
    mesh = plsc.VectorSubcoreMesh(core_axis_name="c", subcore_axis_name="s")
    f = pl.kernel(
        _sc_body,
        out_type=(
            jax.ShapeDtypeStruct((NC, N, H), jnp.float32),
            jax.ShapeDtypeStruct((N, LANES), jnp.float32),
            jax.ShapeDtypeStruct((E, U), jnp.float32),
        ),
        mesh=mesh,
        compiler_params=pltpu.CompilerParams(use_tc_tiling_on_sc=False),
        scratch_types=[
            pltpu.VMEM((2, CHUNK), jnp.int32),
            pltpu.VMEM((2, CHUNK), jnp.int32),
            pltpu.VMEM((2 * CHUNK, H), jnp.float32),
            pltpu.VMEM((2 * CHUNK, H), jnp.float32),
            pltpu.VMEM((2 * CHUNK, H), jnp.float32),
            pltpu.VMEM((2 * CHUNK, H), jnp.float32),
            pltpu.VMEM((CHUNK, LANES), jnp.float32),
            pltpu.VMEM((ZROWS, H), jnp.float32),
            pltpu.VMEM((ZROWS, LANES), jnp.float32),
            pltpu.VMEM_SHARED((N, H), jnp.float32),
            pltpu.VMEM_SHARED((N, LANES), jnp.float32),
            pltpu.SemaphoreType.DMA((2,)),
            pltpu.SemaphoreType.DMA((2,)),
            pltpu.SemaphoreType.DMA((2,)),
            pltpu.SemaphoreType.DMA((2,)),
            pltpu.SemaphoreType.DMA((2,)),
            pltpu.SemaphoreType.DMA((2,)),
        ],
    )
    return f(src, dst, w0, x2s, x3s, x4s)


def _node_mm_body(x_ref, wt_ref, b_ref, o_ref):
    o_ref[...] = (
        jnp.dot(x_ref[...], wt_ref[...], preferred_element_type=jnp.float32)
        + b_ref[...]
    )


def _node_out_body(x0_ref, x1_ref, segp_ref, cnt_ref, g_ref, b_ref, o_ref):
    seg = jnp.concatenate([segp_ref[0], segp_ref[1]], axis=1)
    cnt = jnp.sum(cnt_ref[...], axis=1, keepdims=True) * (1.0 / LANES)
    pooled = seg / jnp.maximum(cnt, 1.0)
    h = x1_ref[...] + pooled
    mu = jnp.mean(h, axis=0, keepdims=True)
    d = h - mu
    var = jnp.mean(d * d, axis=0, keepdims=True)
    z = g_ref[...] * d * lax.rsqrt(var + 1e-5) + b_ref[...]
    o_ref[...] = x0_ref[...] + z * _sigmoid(z)


EB = 3200


def _edge_stats_body(w0_ref, g34_ref, wet_ref, be_ref, s_ref, q_ref):
    ep = (
        jnp.dot(w0_ref[...], wet_ref[...], preferred_element_type=jnp.float32)
        + be_ref[...]
        + g34_ref[...]
    )
    bs = jnp.sum(ep, axis=0, keepdims=True)
    bq = jnp.sum(ep * ep, axis=0, keepdims=True)

    @pl.when(pl.program_id(0) == 0)
    def _():
        s_ref[...] = bs
        q_ref[...] = bq

    @pl.when(pl.program_id(0) > 0)
    def _():
        s_ref[...] += bs
        q_ref[...] += bq


def _edge_out_body(w0_ref, g34_ref, wet_ref, be_ref, s_ref, q_ref,
                   g_ref, b_ref, o_ref):
    ep = (
        jnp.dot(w0_ref[...], wet_ref[...], preferred_element_type=jnp.float32)
        + be_ref[...]
        + g34_ref[...]
    )
    inv_e = 1.0 / E
    mu = s_ref[...] * inv_e
    var = q_ref[...] * inv_e - mu * mu
    z = g_ref[...] * (ep - mu) * lax.rsqrt(var + 1e-5) + b_ref[...]
    o_ref[...] = w0_ref[...] + z * _sigmoid(z)


def kernel(x, edge_index, edge_attr, w_v1, b_v1, w_v2, b_v2, w_v3, b_v3,
           w_v4, b_v4, w_e, b_e, bn_v_gamma, bn_v_beta, bn_e_gamma, bn_e_beta):
    src = edge_index[0]
    dst = edge_index[1]

    wt = jnp.concatenate([w_v1.T, w_v2.T, w_v3.T, w_v4.T], axis=1)
    bc = jnp.concatenate([b_v1, b_v2, b_v3, b_v4]).reshape(1, 4 * U)
    x1234 = pl.pallas_call(
        _node_mm_body,
        out_shape=jax.ShapeDtypeStruct((N, 4 * U), jnp.float32),
    )(x, wt, bc)
    x1 = x1234[:, :U]

    def _halves(a):
        return jnp.stack([a[:, :H], a[:, H:]])

    x2s = _halves(x1234[:, U:2 * U])
    x3s = _halves(x1234[:, 2 * U:3 * U])
    x4s = _halves(x1234[:, 3 * U:])

    seg_parts, cnt16, g34 = _sc_call(src, dst, edge_attr, x2s, x3s, x4s)

    x_out = pl.pallas_call(
        _node_out_body,
        out_shape=jax.ShapeDtypeStruct((N, U), jnp.float32),
    )(x, x1, seg_parts, cnt16,
      bn_v_gamma.reshape(1, U), bn_v_beta.reshape(1, U))

    grid = E // EB
    ssum, ssq = pl.pallas_call(
        _edge_stats_body,
        grid=(grid,),
        in_specs=[
            pl.BlockSpec((EB, U), lambda i: (i, 0)),
            pl.BlockSpec((EB, U), lambda i: (i, 0)),
            pl.BlockSpec((U, U), lambda i: (0, 0)),
            pl.BlockSpec((1, U), lambda i: (0, 0)),
        ],
        out_specs=[
            pl.BlockSpec((1, U), lambda i: (0, 0)),
            pl.BlockSpec((1, U), lambda i: (0, 0)),
        ],
        out_shape=[
            jax.ShapeDtypeStruct((1, U), jnp.float32),
            jax.ShapeDtypeStruct((1, U), jnp.float32),
        ],
    )(edge_attr, g34, w_e.T, b_e.reshape(1, U))

    w_out = pl.pallas_call(
        _edge_out_body,
        grid=(grid,),
        in_specs=[
            pl.BlockSpec((EB, U), lambda i: (i, 0)),
            pl.BlockSpec((EB, U), lambda i: (i, 0)),
            pl.BlockSpec((U, U), lambda i: (0, 0)),
            pl.BlockSpec((1, U), lambda i: (0, 0)),
            pl.BlockSpec((1, U), lambda i: (0, 0)),
            pl.BlockSpec((1, U), lambda i: (0, 0)),
            pl.BlockSpec((1, U), lambda i: (0, 0)),
            pl.BlockSpec((1, U), lambda i: (0, 0)),
        ],
        out_specs=pl.BlockSpec((EB, U), lambda i: (i, 0)),
        out_shape=jax.ShapeDtypeStruct((E, U), jnp.float32),
    )(edge_attr, g34, w_e.T, b_e.reshape(1, U), ssum, ssq,
      bn_e_gamma.reshape(1, U), bn_e_beta.reshape(1, U))

    return (x_out, w_out)


# speedup vs baseline: 3.2560x; 1.1566x over previous
import jax
import jax.numpy as jnp
from jax import lax
from jax.experimental import pallas as pl
from jax.experimental.pallas import tpu as pltpu
from jax.experimental.pallas import tpu_sc as plsc

N = 10000
E = 320000
U = 128

NC = 2
NS = 16
LANES = 16

H = U // NC
CHUNK = 128
NCHUNKS = E // CHUNK
TRIPS = (NCHUNKS + NS - 1) // NS
ROWS_PER_TILE = N // NS
ZROWS = 125


def _sigmoid(v):
    return 1.0 / (1.0 + jnp.exp(-v))


def _sc_body(src_hbm, dst_hbm, w0_hbm, x2_hbm, x3_hbm, x4_hbm,
             seg_hbm, cnt_hbm, g34_hbm,
             idx_src, idx_dst, w0c, x2r, g3, g4, ones16, zbuf, zcnt,
             seg_acc, cnt_acc,
             sem_i, sem_g2, sem_g34, sem_s, sem_c, sem_o):
    cid = lax.axis_index("c")
    sid = lax.axis_index("s")

    zeros16 = jnp.zeros((LANES,), jnp.float32)
    ones = jnp.ones((LANES,), jnp.float32)

    def init_ones(i, _):
        ones16[i, :] = ones
        return 0
    lax.fori_loop(0, CHUNK, init_ones, 0)

    def init_zbuf(i, _):
        for j in range(H // LANES):
            zbuf[i, pl.ds(j * LANES, LANES)] = zeros16
        zcnt[i, :] = zeros16
        return 0
    lax.fori_loop(0, ZROWS, init_zbuf, 0)

    for kk in range(ROWS_PER_TILE // ZROWS):
        off = sid * ROWS_PER_TILE + kk * ZROWS
        pltpu.sync_copy(zbuf, seg_acc.at[pl.ds(off, ZROWS)])
        pltpu.sync_copy(zcnt, cnt_acc.at[pl.ds(off, ZROWS)])
    plsc.subcore_barrier()

    def d_src(q, k):
        return pltpu.make_async_copy(
            src_hbm.at[pl.ds(k * CHUNK, CHUNK)], idx_src.at[q], sem_i.at[q])

    def d_dst(q, k):
        return pltpu.make_async_copy(
            dst_hbm.at[pl.ds(k * CHUNK, CHUNK)], idx_dst.at[q], sem_i.at[q])

    def d_w0(q, k):
        return pltpu.make_async_copy(
            w0_hbm.at[pl.ds(k * CHUNK, CHUNK), pl.ds(cid * H, H)],
            w0c.at[pl.ds(q * CHUNK, CHUNK)], sem_i.at[q])

    def d_x2(q):
        return pltpu.make_async_copy(
            x2_hbm.at[cid].at[idx_dst.at[q]],
            x2r.at[pl.ds(q * CHUNK, CHUNK)], sem_g2.at[q])

    def d_x3(q):
        return pltpu.make_async_copy(
            x3_hbm.at[cid].at[idx_src.at[q]],
            g3.at[pl.ds(q * CHUNK, CHUNK)], sem_g34.at[q])

    def d_x4(q):
        return pltpu.make_async_copy(
            x4_hbm.at[cid].at[idx_dst.at[q]],
            g4.at[pl.ds(q * CHUNK, CHUNK)], sem_g34.at[q])

    def d_seg(q):
        return pltpu.make_async_copy(
            x2r.at[pl.ds(q * CHUNK, CHUNK)],
            seg_acc.at[idx_src.at[q]], sem_s.at[q])

    def d_cnt(q):
        return pltpu.make_async_copy(
            ones16, cnt_acc.at[idx_src.at[q]], sem_c.at[q])

    def d_g34(q, k):
        return pltpu.make_async_copy(
            g3.at[pl.ds(q * CHUNK, CHUNK)],
            g34_hbm.at[pl.ds(k * CHUNK, CHUNK), pl.ds(cid * H, H)],
            sem_o.at[q])

    d_src(0, sid).start()
    d_dst(0, sid).start()
    d_w0(0, sid).start()

    def chunk(q, t):
        nq = 1 - q
        k = sid + t * NS

        @pl.when(k < NCHUNKS)
        def _():
            d_src(q, k).wait()
            d_dst(q, k).wait()
            d_w0(q, k).wait()

            d_x2(q).start()
            d_x3(q).start()
            d_x4(q).start()

            @pl.when(t >= 1)
            def _():
                d_seg(nq).wait()

                @pl.when(cid == 0)
                def _():
                    d_cnt(nq).wait()
                d_g34(nq, k - NS).wait()

            @pl.when(k + NS < NCHUNKS)
            def _():
                d_src(nq, k + NS).start()
                d_dst(nq, k + NS).start()
                d_w0(nq, k + NS).start()

            d_x2(q).wait()
            row0 = q * CHUNK

            def msg_row(i, _):
                for j in range(H // LANES):
                    sl = pl.ds(j * LANES, LANES)
                    x2r[row0 + i, sl] = x2r[row0 + i, sl] * _sigmoid(
                        w0c[row0 + i, sl])
                return 0
            lax.fori_loop(0, CHUNK, msg_row, 0)

            d_seg(q).start(add=True)

            @pl.when(cid == 0)
            def _():
                d_cnt(q).start(add=True)

            d_x3(q).wait()
            d_x4(q).wait()

            def add_row(i, _):
                for j in range(H // LANES):
                    sl = pl.ds(j * LANES, LANES)
                    g3[row0 + i, sl] = g3[row0 + i, sl] + g4[row0 + i, sl]
                return 0
            lax.fori_loop(0, CHUNK, add_row, 0)
            d_g34(q, k).start()

    def pair(tt, _):
        chunk(0, 2 * tt)
        chunk(1, 2 * tt + 1)
        return 0

    lax.fori_loop(0, (TRIPS + 1) // 2, pair, 0)

    t_last0 = TRIPS - 1
    ran_last0 = sid + t_last0 * NS < NCHUNKS

    @pl.when(ran_last0)
    def _():
        d_seg(0).wait()

        @pl.when(cid == 0)
        def _():
            d_cnt(0).wait()
        d_g34(0, sid + t_last0 * NS).wait()

    @pl.when(jnp.logical_not(ran_last0))
    def _():
        d_seg(1).wait()

        @pl.when(cid == 0)
        def _():
            d_cnt(1).wait()
        d_g34(1, sid + (TRIPS - 2) * NS).wait()

    plsc.subcore_barrier()

    @pl.when(sid == 0)
    def _():
        pltpu.sync_copy(seg_acc, seg_hbm.at[cid])

        @pl.when(cid == 0)
        def _():
            pltpu.sync_copy(cnt_acc, cnt_hbm)


def _sc_call(src, dst, w0, x2s, x3s, x4s):
    mesh = plsc.VectorSubcoreMesh(core_axis_name="c", subcore_axis_name="s")
    f = pl.kernel(
        _sc_body,
        out_type=(
            jax.ShapeDtypeStruct((NC, N, H), jnp.float32),
            jax.ShapeDtypeStruct((N, LANES), jnp.float32),
            jax.ShapeDtypeStruct((E, U), jnp.float32),
        ),
        mesh=mesh,
        compiler_params=pltpu.CompilerParams(use_tc_tiling_on_sc=False),
        scratch_types=[
            pltpu.VMEM((2, CHUNK), jnp.int32),
            pltpu.VMEM((2, CHUNK), jnp.int32),
            pltpu.VMEM((2 * CHUNK, H), jnp.float32),
            pltpu.VMEM((2 * CHUNK, H), jnp.float32),
            pltpu.VMEM((2 * CHUNK, H), jnp.float32),
            pltpu.VMEM((2 * CHUNK, H), jnp.float32),
            pltpu.VMEM((CHUNK, LANES), jnp.float32),
            pltpu.VMEM((ZROWS, H), jnp.float32),
            pltpu.VMEM((ZROWS, LANES), jnp.float32),
            pltpu.VMEM_SHARED((N, H), jnp.float32),
            pltpu.VMEM_SHARED((N, LANES), jnp.float32),
            pltpu.SemaphoreType.DMA((2,)),
            pltpu.SemaphoreType.DMA((2,)),
            pltpu.SemaphoreType.DMA((2,)),
            pltpu.SemaphoreType.DMA((2,)),
            pltpu.SemaphoreType.DMA((2,)),
            pltpu.SemaphoreType.DMA((2,)),
        ],
    )
    return f(src, dst, w0, x2s, x3s, x4s)


def _node_mm_body(x_ref, wt_ref, b_ref, o_ref):
    o_ref[...] = (
        jnp.dot(x_ref[...], wt_ref[...], preferred_element_type=jnp.float32)
        + b_ref[...]
    )


def _node_out_body(x0_ref, x1_ref, segp_ref, cnt_ref, g_ref, b_ref, o_ref):
    seg = jnp.concatenate([segp_ref[0], segp_ref[1]], axis=1)
    cnt = jnp.sum(cnt_ref[...], axis=1, keepdims=True) * (1.0 / LANES)
    pooled = seg / jnp.maximum(cnt, 1.0)
    h = x1_ref[...] + pooled
    mu = jnp.mean(h, axis=0, keepdims=True)
    d = h - mu
    var = jnp.mean(d * d, axis=0, keepdims=True)
    z = g_ref[...] * d * lax.rsqrt(var + 1e-5) + b_ref[...]
    o_ref[...] = x0_ref[...] + z * _sigmoid(z)


EB = 3200


def _edge_stats_body(w0_ref, g34_ref, wet_ref, be_ref, s_ref, q_ref):
    ep = (
        jnp.dot(w0_ref[...], wet_ref[...], preferred_element_type=jnp.float32)
        + be_ref[...]
        + g34_ref[...]
    )
    bs = jnp.sum(ep, axis=0, keepdims=True)
    bq = jnp.sum(ep * ep, axis=0, keepdims=True)

    @pl.when(pl.program_id(0) == 0)
    def _():
        s_ref[...] = bs
        q_ref[...] = bq

    @pl.when(pl.program_id(0) > 0)
    def _():
        s_ref[...] += bs
        q_ref[...] += bq


def _edge_out_body(w0_ref, g34_ref, wet_ref, be_ref, s_ref, q_ref,
                   g_ref, b_ref, o_ref):
    ep = (
        jnp.dot(w0_ref[...], wet_ref[...], preferred_element_type=jnp.float32)
        + be_ref[...]
        + g34_ref[...]
    )
    inv_e = 1.0 / E
    mu = s_ref[...] * inv_e
    var = q_ref[...] * inv_e - mu * mu
    z = g_ref[...] * (ep - mu) * lax.rsqrt(var + 1e-5) + b_ref[...]
    o_ref[...] = w0_ref[...] + z * _sigmoid(z)


def kernel(x, edge_index, edge_attr, w_v1, b_v1, w_v2, b_v2, w_v3, b_v3,
           w_v4, b_v4, w_e, b_e, bn_v_gamma, bn_v_beta, bn_e_gamma, bn_e_beta):
    src = edge_index[0]
    dst = edge_index[1]

    wt = jnp.concatenate([w_v1.T, w_v2.T, w_v3.T, w_v4.T], axis=1)
    bc = jnp.concatenate([b_v1, b_v2, b_v3, b_v4]).reshape(1, 4 * U)
    x1234 = pl.pallas_call(
        _node_mm_body,
        out_shape=jax.ShapeDtypeStruct((N, 4 * U), jnp.float32),
    )(x, wt, bc)
    x1 = x1234[:, :U]

    def _halves(a):
        return jnp.stack([a[:, :H], a[:, H:]])

    x2s = _halves(x1234[:, U:2 * U])
    x3s = _halves(x1234[:, 2 * U:3 * U])
    x4s = _halves(x1234[:, 3 * U:])

    seg_parts, cnt16, g34 = _sc_call(src, dst, edge_attr, x2s, x3s, x4s)

    x_out = pl.pallas_call(
        _node_out_body,
        out_shape=jax.ShapeDtypeStruct((N, U), jnp.float32),
    )(x, x1, seg_parts, cnt16,
      bn_v_gamma.reshape(1, U), bn_v_beta.reshape(1, U))

    grid = E // EB
    ssum, ssq = pl.pallas_call(
        _edge_stats_body,
        grid=(grid,),
        in_specs=[
            pl.BlockSpec((EB, U), lambda i: (i, 0)),
            pl.BlockSpec((EB, U), lambda i: (i, 0)),
            pl.BlockSpec((U, U), lambda i: (0, 0)),
            pl.BlockSpec((1, U), lambda i: (0, 0)),
        ],
        out_specs=[
            pl.BlockSpec((1, U), lambda i: (0, 0)),
            pl.BlockSpec((1, U), lambda i: (0, 0)),
        ],
        out_shape=[
            jax.ShapeDtypeStruct((1, U), jnp.float32),
            jax.ShapeDtypeStruct((1, U), jnp.float32),
        ],
    )(edge_attr, g34, w_e.T, b_e.reshape(1, U))

    w_out = pl.pallas_call(
        _edge_out_body,
        grid=(grid,),
        in_specs=[
            pl.BlockSpec((EB, U), lambda i: (i, 0)),
            pl.BlockSpec((EB, U), lambda i: (i, 0)),
            pl.BlockSpec((U, U), lambda i: (0, 0)),
            pl.BlockSpec((1, U), lambda i: (0, 0)),
            pl.BlockSpec((1, U), lambda i: (0, 0)),
            pl.BlockSpec((1, U), lambda i: (0, 0)),
            pl.BlockSpec((1, U), lambda i: (0, 0)),
            pl.BlockSpec((1, U), lambda i: (0, 0)),
        ],
        out_specs=pl.BlockSpec((EB, U), lambda i: (i, 0)),
        out_shape=jax.ShapeDtypeStruct((E, U), jnp.float32),
    )(edge_attr, g34, w_e.T, b_e.reshape(1, U), ssum, ssq,
      bn_e_gamma.reshape(1, U), bn_e_beta.reshape(1, U))

    return (x_out, w_out)


# speedup vs baseline: 3.4561x; 1.0615x over previous
import jax
import jax.numpy as jnp
from jax import lax
from jax.experimental import pallas as pl
from jax.experimental.pallas import tpu as pltpu
from jax.experimental.pallas import tpu_sc as plsc

N = 10000
E = 320000
U = 128

NC = 2
NS = 16
LANES = 16

H = U // NC
CHUNK = 128
NCHUNKS = E // CHUNK
TRIPS = (NCHUNKS + NS - 1) // NS
ROWS_PER_TILE = N // NS
ZROWS = 125


def _sigmoid(v):
    return 1.0 / (1.0 + jnp.exp(-v))


def _sc_body(src_hbm, dst_hbm, w0_hbm, x2_hbm, x3_hbm, x4_hbm,
             seg_hbm, cnt_hbm, g34_hbm,
             idx_src, idx_dst, w0c, x2r, g3, g4, ones16, zbuf, zcnt,
             seg_acc, cnt_acc,
             sem_i, sem_g2, sem_g34, sem_s, sem_c, sem_o):
    cid = lax.axis_index("c")
    sid = lax.axis_index("s")

    zeros16 = jnp.zeros((LANES,), jnp.float32)
    ones = jnp.ones((LANES,), jnp.float32)

    def init_ones(i, _):
        ones16[i, :] = ones
        return 0
    lax.fori_loop(0, CHUNK, init_ones, 0)

    def init_zbuf(i, _):
        for j in range(H // LANES):
            zbuf[i, pl.ds(j * LANES, LANES)] = zeros16
        zcnt[i, :] = zeros16
        return 0
    lax.fori_loop(0, ZROWS, init_zbuf, 0)

    for kk in range(ROWS_PER_TILE // ZROWS):
        off = sid * ROWS_PER_TILE + kk * ZROWS
        pltpu.sync_copy(zbuf, seg_acc.at[pl.ds(off, ZROWS)])
        pltpu.sync_copy(zcnt, cnt_acc.at[pl.ds(off, ZROWS)])
    plsc.subcore_barrier()

    def d_src(q, k):
        return pltpu.make_async_copy(
            src_hbm.at[pl.ds(k * CHUNK, CHUNK)], idx_src.at[q], sem_i.at[q])

    def d_dst(q, k):
        return pltpu.make_async_copy(
            dst_hbm.at[pl.ds(k * CHUNK, CHUNK)], idx_dst.at[q], sem_i.at[q])

    def d_w0(q, k):
        return pltpu.make_async_copy(
            w0_hbm.at[pl.ds(k * CHUNK, CHUNK), pl.ds(cid * H, H)],
            w0c.at[pl.ds(q * CHUNK, CHUNK)], sem_i.at[q])

    def d_x2(q):
        return pltpu.make_async_copy(
            x2_hbm.at[cid].at[idx_dst.at[q]],
            x2r.at[pl.ds(q * CHUNK, CHUNK)], sem_g2.at[q])

    def d_x3(q):
        return pltpu.make_async_copy(
            x3_hbm.at[cid].at[idx_src.at[q]],
            g3.at[pl.ds(q * CHUNK, CHUNK)], sem_g34.at[q])

    def d_x4(q):
        return pltpu.make_async_copy(
            x4_hbm.at[cid].at[idx_dst.at[q]],
            g4.at[pl.ds(q * CHUNK, CHUNK)], sem_g34.at[q])

    def d_seg(q):
        return pltpu.make_async_copy(
            x2r.at[pl.ds(q * CHUNK, CHUNK)],
            seg_acc.at[idx_src.at[q]], sem_s.at[q])

    def d_cnt(q):
        return pltpu.make_async_copy(
            ones16, cnt_acc.at[idx_src.at[q]], sem_c.at[q])

    def d_g34(q, k):
        return pltpu.make_async_copy(
            g3.at[pl.ds(q * CHUNK, CHUNK)],
            g34_hbm.at[pl.ds(k * CHUNK, CHUNK), pl.ds(cid * H, H)],
            sem_o.at[q])

    d_src(0, sid).start()
    d_dst(0, sid).start()
    d_w0(0, sid).start()

    def chunk(q, t):
        nq = 1 - q
        k = sid + t * NS

        @pl.when(k < NCHUNKS)
        def _():
            d_src(q, k).wait()
            d_dst(q, k).wait()
            d_w0(q, k).wait()

            d_x2(q).start()
            d_x3(q).start()
            d_x4(q).start()

            @pl.when(t >= 1)
            def _():
                d_seg(nq).wait()

                @pl.when(cid == 0)
                def _():
                    d_cnt(nq).wait()
                d_g34(nq, k - NS).wait()

            @pl.when(k + NS < NCHUNKS)
            def _():
                d_src(nq, k + NS).start()
                d_dst(nq, k + NS).start()
                d_w0(nq, k + NS).start()

            d_x2(q).wait()
            row0 = q * CHUNK

            def msg_row(i, _):
                for j in range(H // LANES):
                    sl = pl.ds(j * LANES, LANES)
                    x2r[row0 + i, sl] = x2r[row0 + i, sl] * _sigmoid(
                        w0c[row0 + i, sl])
                return 0
            lax.fori_loop(0, CHUNK, msg_row, 0)

            d_seg(q).start(add=True)

            @pl.when(cid == 0)
            def _():
                d_cnt(q).start(add=True)

            d_x3(q).wait()
            d_x4(q).wait()

            def add_row(i, _):
                for j in range(H // LANES):
                    sl = pl.ds(j * LANES, LANES)
                    g3[row0 + i, sl] = g3[row0 + i, sl] + g4[row0 + i, sl]
                return 0
            lax.fori_loop(0, CHUNK, add_row, 0)
            d_g34(q, k).start()

    def pair(tt, _):
        chunk(0, 2 * tt)
        chunk(1, 2 * tt + 1)
        return 0

    lax.fori_loop(0, (TRIPS + 1) // 2, pair, 0)

    t_last0 = TRIPS - 1
    ran_last0 = sid + t_last0 * NS < NCHUNKS

    @pl.when(ran_last0)
    def _():
        d_seg(0).wait()

        @pl.when(cid == 0)
        def _():
            d_cnt(0).wait()
        d_g34(0, sid + t_last0 * NS).wait()

    @pl.when(jnp.logical_not(ran_last0))
    def _():
        d_seg(1).wait()

        @pl.when(cid == 0)
        def _():
            d_cnt(1).wait()
        d_g34(1, sid + (TRIPS - 2) * NS).wait()

    plsc.subcore_barrier()

    @pl.when(sid == 0)
    def _():
        pltpu.sync_copy(seg_acc, seg_hbm.at[cid])

        @pl.when(cid == 0)
        def _():
            pltpu.sync_copy(cnt_acc, cnt_hbm)


def _sc_call(src, dst, w0, x2s, x3s, x4s):
    mesh = plsc.VectorSubcoreMesh(core_axis_name="c", subcore_axis_name="s")
    f = pl.kernel(
        _sc_body,
        out_type=(
            jax.ShapeDtypeStruct((NC, N, H), jnp.float32),
            jax.ShapeDtypeStruct((N, LANES), jnp.float32),
            jax.ShapeDtypeStruct((E, U), jnp.float32),
        ),
        mesh=mesh,
        compiler_params=pltpu.CompilerParams(use_tc_tiling_on_sc=False),
        scratch_types=[
            pltpu.VMEM((2, CHUNK), jnp.int32),
            pltpu.VMEM((2, CHUNK), jnp.int32),
            pltpu.VMEM((2 * CHUNK, H), jnp.float32),
            pltpu.VMEM((2 * CHUNK, H), jnp.float32),
            pltpu.VMEM((2 * CHUNK, H), jnp.float32),
            pltpu.VMEM((2 * CHUNK, H), jnp.float32),
            pltpu.VMEM((CHUNK, LANES), jnp.float32),
            pltpu.VMEM((ZROWS, H), jnp.float32),
            pltpu.VMEM((ZROWS, LANES), jnp.float32),
            pltpu.VMEM_SHARED((N, H), jnp.float32),
            pltpu.VMEM_SHARED((N, LANES), jnp.float32),
            pltpu.SemaphoreType.DMA((2,)),
            pltpu.SemaphoreType.DMA((2,)),
            pltpu.SemaphoreType.DMA((2,)),
            pltpu.SemaphoreType.DMA((2,)),
            pltpu.SemaphoreType.DMA((2,)),
            pltpu.SemaphoreType.DMA((2,)),
        ],
    )
    return f(src, dst, w0, x2s, x3s, x4s)


def _node_mm_body(x_ref, wt_ref, b_ref, o_ref):
    o_ref[...] = (
        jnp.dot(x_ref[...], wt_ref[...], preferred_element_type=jnp.float32)
        + b_ref[...]
    )


def _node_out_body(x0_ref, x1_ref, segp_ref, cnt_ref, g_ref, b_ref, o_ref):
    seg = jnp.concatenate([segp_ref[0], segp_ref[1]], axis=1)
    cnt = jnp.sum(cnt_ref[...], axis=1, keepdims=True) * (1.0 / LANES)
    pooled = seg / jnp.maximum(cnt, 1.0)
    h = x1_ref[...] + pooled
    mu = jnp.mean(h, axis=0, keepdims=True)
    d = h - mu
    var = jnp.mean(d * d, axis=0, keepdims=True)
    z = g_ref[...] * d * lax.rsqrt(var + 1e-5) + b_ref[...]
    o_ref[...] = x0_ref[...] + z * _sigmoid(z)


EB = 6400


def _edge_stats_body(w0_ref, g34_ref, wet_ref, be_ref, s_ref, q_ref):
    ep = (
        jnp.dot(w0_ref[...], wet_ref[...], preferred_element_type=jnp.float32)
        + be_ref[...]
        + g34_ref[...]
    )
    bs = jnp.sum(ep, axis=0, keepdims=True)
    bq = jnp.sum(ep * ep, axis=0, keepdims=True)

    @pl.when(pl.program_id(0) == 0)
    def _():
        s_ref[...] = bs
        q_ref[...] = bq

    @pl.when(pl.program_id(0) > 0)
    def _():
        s_ref[...] += bs
        q_ref[...] += bq


def _edge_out_body(w0_ref, g34_ref, wet_ref, be_ref, s_ref, q_ref,
                   g_ref, b_ref, o_ref):
    ep = (
        jnp.dot(w0_ref[...], wet_ref[...], preferred_element_type=jnp.float32)
        + be_ref[...]
        + g34_ref[...]
    )
    inv_e = 1.0 / E
    mu = s_ref[...] * inv_e
    var = q_ref[...] * inv_e - mu * mu
    z = g_ref[...] * (ep - mu) * lax.rsqrt(var + 1e-5) + b_ref[...]
    o_ref[...] = w0_ref[...] + z * _sigmoid(z)


def kernel(x, edge_index, edge_attr, w_v1, b_v1, w_v2, b_v2, w_v3, b_v3,
           w_v4, b_v4, w_e, b_e, bn_v_gamma, bn_v_beta, bn_e_gamma, bn_e_beta):
    src = edge_index[0]
    dst = edge_index[1]

    wt = jnp.concatenate([w_v1.T, w_v2.T, w_v3.T, w_v4.T], axis=1)
    bc = jnp.concatenate([b_v1, b_v2, b_v3, b_v4]).reshape(1, 4 * U)
    x1234 = pl.pallas_call(
        _node_mm_body,
        out_shape=jax.ShapeDtypeStruct((N, 4 * U), jnp.float32),
    )(x, wt, bc)
    x1 = x1234[:, :U]

    def _halves(a):
        return jnp.stack([a[:, :H], a[:, H:]])

    x2s = _halves(x1234[:, U:2 * U])
    x3s = _halves(x1234[:, 2 * U:3 * U])
    x4s = _halves(x1234[:, 3 * U:])

    seg_parts, cnt16, g34 = _sc_call(src, dst, edge_attr, x2s, x3s, x4s)

    x_out = pl.pallas_call(
        _node_out_body,
        out_shape=jax.ShapeDtypeStruct((N, U), jnp.float32),
    )(x, x1, seg_parts, cnt16,
      bn_v_gamma.reshape(1, U), bn_v_beta.reshape(1, U))

    grid = E // EB
    ssum, ssq = pl.pallas_call(
        _edge_stats_body,
        grid=(grid,),
        in_specs=[
            pl.BlockSpec((EB, U), lambda i: (i, 0)),
            pl.BlockSpec((EB, U), lambda i: (i, 0)),
            pl.BlockSpec((U, U), lambda i: (0, 0)),
            pl.BlockSpec((1, U), lambda i: (0, 0)),
        ],
        out_specs=[
            pl.BlockSpec((1, U), lambda i: (0, 0)),
            pl.BlockSpec((1, U), lambda i: (0, 0)),
        ],
        out_shape=[
            jax.ShapeDtypeStruct((1, U), jnp.float32),
            jax.ShapeDtypeStruct((1, U), jnp.float32),
        ],
    )(edge_attr, g34, w_e.T, b_e.reshape(1, U))

    w_out = pl.pallas_call(
        _edge_out_body,
        grid=(grid,),
        in_specs=[
            pl.BlockSpec((EB, U), lambda i: (i, 0)),
            pl.BlockSpec((EB, U), lambda i: (i, 0)),
            pl.BlockSpec((U, U), lambda i: (0, 0)),
            pl.BlockSpec((1, U), lambda i: (0, 0)),
            pl.BlockSpec((1, U), lambda i: (0, 0)),
            pl.BlockSpec((1, U), lambda i: (0, 0)),
            pl.BlockSpec((1, U), lambda i: (0, 0)),
            pl.BlockSpec((1, U), lambda i: (0, 0)),
        ],
        out_specs=pl.BlockSpec((EB, U), lambda i: (i, 0)),
        out_shape=jax.ShapeDtypeStruct((E, U), jnp.float32),
    )(edge_attr, g34, w_e.T, b_e.reshape(1, U), ssum, ssq,
      bn_e_gamma.reshape(1, U), bn_e_beta.reshape(1, U))

    return (x_out, w_out)


# speedup vs baseline: 3.4685x; 1.0036x over previous
import jax
import jax.numpy as jnp
from jax import lax
from jax.experimental import pallas as pl
from jax.experimental.pallas import tpu as pltpu
from jax.experimental.pallas import tpu_sc as plsc

N = 10000
E = 320000
U = 128

NC = 2
NS = 16
LANES = 16

H = U // NC
CHUNK = 128
NCHUNKS = E // CHUNK
TRIPS = (NCHUNKS + NS - 1) // NS
ROWS_PER_TILE = N // NS
ZROWS = 125


def _sigmoid(v):
    return 1.0 / (1.0 + jnp.exp(-v))


def _sc_body(src_hbm, dst_hbm, w0_hbm, x2_hbm, x3_hbm, x4_hbm,
             seg_hbm, cnt_hbm, g34_hbm,
             idx_src, idx_dst, w0c, x2r, g3, g4, ones16, zbuf, zcnt,
             seg_acc, cnt_acc,
             sem_i, sem_g2, sem_g34, sem_s, sem_c, sem_o):
    cid = lax.axis_index("c")
    sid = lax.axis_index("s")

    zeros16 = jnp.zeros((LANES,), jnp.float32)
    ones = jnp.ones((LANES,), jnp.float32)

    def init_ones(i, _):
        ones16[i, :] = ones
        return 0
    lax.fori_loop(0, CHUNK, init_ones, 0)

    def init_zbuf(i, _):
        for j in range(H // LANES):
            zbuf[i, pl.ds(j * LANES, LANES)] = zeros16
        zcnt[i, :] = zeros16
        return 0
    lax.fori_loop(0, ZROWS, init_zbuf, 0)

    for kk in range(ROWS_PER_TILE // ZROWS):
        off = sid * ROWS_PER_TILE + kk * ZROWS
        pltpu.sync_copy(zbuf, seg_acc.at[pl.ds(off, ZROWS)])
        pltpu.sync_copy(zcnt, cnt_acc.at[pl.ds(off, ZROWS)])
    plsc.subcore_barrier()

    def d_src(q, k):
        return pltpu.make_async_copy(
            src_hbm.at[pl.ds(k * CHUNK, CHUNK)], idx_src.at[q], sem_i.at[q])

    def d_dst(q, k):
        return pltpu.make_async_copy(
            dst_hbm.at[pl.ds(k * CHUNK, CHUNK)], idx_dst.at[q], sem_i.at[q])

    def d_w0(q, k):
        return pltpu.make_async_copy(
            w0_hbm.at[pl.ds(k * CHUNK, CHUNK), pl.ds(cid * H, H)],
            w0c.at[pl.ds(q * CHUNK, CHUNK)], sem_i.at[q])

    def d_x2(q):
        return pltpu.make_async_copy(
            x2_hbm.at[cid].at[idx_dst.at[q]],
            x2r.at[pl.ds(q * CHUNK, CHUNK)], sem_g2.at[q])

    def d_x3(q):
        return pltpu.make_async_copy(
            x3_hbm.at[cid].at[idx_src.at[q]],
            g3.at[pl.ds(q * CHUNK, CHUNK)], sem_g34.at[q])

    def d_x4(q):
        return pltpu.make_async_copy(
            x4_hbm.at[cid].at[idx_dst.at[q]],
            g4.at[pl.ds(q * CHUNK, CHUNK)], sem_g34.at[q])

    def d_seg(q):
        return pltpu.make_async_copy(
            x2r.at[pl.ds(q * CHUNK, CHUNK)],
            seg_acc.at[idx_src.at[q]], sem_s.at[q])

    def d_cnt(q):
        return pltpu.make_async_copy(
            ones16, cnt_acc.at[idx_src.at[q]], sem_c.at[q])

    def d_g34(q, k):
        return pltpu.make_async_copy(
            g3.at[pl.ds(q * CHUNK, CHUNK)],
            g34_hbm.at[pl.ds(k * CHUNK, CHUNK), pl.ds(cid * H, H)],
            sem_o.at[q])

    d_src(0, sid).start()
    d_dst(0, sid).start()
    d_w0(0, sid).start()

    def chunk(q, t):
        nq = 1 - q
        k = sid + t * NS

        @pl.when(k < NCHUNKS)
        def _():
            d_src(q, k).wait()
            d_dst(q, k).wait()
            d_w0(q, k).wait()

            d_x2(q).start()
            d_x3(q).start()
            d_x4(q).start()

            @pl.when(t >= 1)
            def _():
                d_seg(nq).wait()

                @pl.when(cid == 0)
                def _():
                    d_cnt(nq).wait()
                d_g34(nq, k - NS).wait()

            @pl.when(k + NS < NCHUNKS)
            def _():
                d_src(nq, k + NS).start()
                d_dst(nq, k + NS).start()
                d_w0(nq, k + NS).start()

            d_x2(q).wait()
            row0 = q * CHUNK

            def msg_row(i, _):
                for j in range(H // LANES):
                    sl = pl.ds(j * LANES, LANES)
                    x2r[row0 + i, sl] = x2r[row0 + i, sl] * _sigmoid(
                        w0c[row0 + i, sl])
                return 0
            lax.fori_loop(0, CHUNK, msg_row, 0)

            d_seg(q).start(add=True)

            @pl.when(cid == 0)
            def _():
                d_cnt(q).start(add=True)

            d_x3(q).wait()
            d_x4(q).wait()

            def add_row(i, _):
                for j in range(H // LANES):
                    sl = pl.ds(j * LANES, LANES)
                    g3[row0 + i, sl] = g3[row0 + i, sl] + g4[row0 + i, sl]
                return 0
            lax.fori_loop(0, CHUNK, add_row, 0)
            d_g34(q, k).start()

    def pair(tt, _):
        chunk(0, 2 * tt)
        chunk(1, 2 * tt + 1)
        return 0

    lax.fori_loop(0, (TRIPS + 1) // 2, pair, 0)

    t_last0 = TRIPS - 1
    ran_last0 = sid + t_last0 * NS < NCHUNKS

    @pl.when(ran_last0)
    def _():
        d_seg(0).wait()

        @pl.when(cid == 0)
        def _():
            d_cnt(0).wait()
        d_g34(0, sid + t_last0 * NS).wait()

    @pl.when(jnp.logical_not(ran_last0))
    def _():
        d_seg(1).wait()

        @pl.when(cid == 0)
        def _():
            d_cnt(1).wait()
        d_g34(1, sid + (TRIPS - 2) * NS).wait()

    plsc.subcore_barrier()

    @pl.when(sid == 0)
    def _():
        pltpu.sync_copy(seg_acc, seg_hbm.at[cid])

        @pl.when(cid == 0)
        def _():
            pltpu.sync_copy(cnt_acc, cnt_hbm)


def _sc_call(src, dst, w0, x2s, x3s, x4s):
    mesh = plsc.VectorSubcoreMesh(core_axis_name="c", subcore_axis_name="s")
    f = pl.kernel(
        _sc_body,
        out_type=(
            jax.ShapeDtypeStruct((NC, N, H), jnp.float32),
            jax.ShapeDtypeStruct((N, LANES), jnp.float32),
            jax.ShapeDtypeStruct((E, U), jnp.float32),
        ),
        mesh=mesh,
        compiler_params=pltpu.CompilerParams(use_tc_tiling_on_sc=False),
        scratch_types=[
            pltpu.VMEM((2, CHUNK), jnp.int32),
            pltpu.VMEM((2, CHUNK), jnp.int32),
            pltpu.VMEM((2 * CHUNK, H), jnp.float32),
            pltpu.VMEM((2 * CHUNK, H), jnp.float32),
            pltpu.VMEM((2 * CHUNK, H), jnp.float32),
            pltpu.VMEM((2 * CHUNK, H), jnp.float32),
            pltpu.VMEM((CHUNK, LANES), jnp.float32),
            pltpu.VMEM((ZROWS, H), jnp.float32),
            pltpu.VMEM((ZROWS, LANES), jnp.float32),
            pltpu.VMEM_SHARED((N, H), jnp.float32),
            pltpu.VMEM_SHARED((N, LANES), jnp.float32),
            pltpu.SemaphoreType.DMA((2,)),
            pltpu.SemaphoreType.DMA((2,)),
            pltpu.SemaphoreType.DMA((2,)),
            pltpu.SemaphoreType.DMA((2,)),
            pltpu.SemaphoreType.DMA((2,)),
            pltpu.SemaphoreType.DMA((2,)),
        ],
    )
    return f(src, dst, w0, x2s, x3s, x4s)


def _node_mm_body(x_ref, wt_ref, b_ref, o_ref):
    o_ref[...] = (
        jnp.dot(x_ref[...], wt_ref[...], preferred_element_type=jnp.float32)
        + b_ref[...]
    )


def _node_out_body(x0_ref, x1_ref, segp_ref, cnt_ref, g_ref, b_ref, o_ref):
    seg = jnp.concatenate([segp_ref[0], segp_ref[1]], axis=1)
    cnt = jnp.sum(cnt_ref[...], axis=1, keepdims=True) * (1.0 / LANES)
    pooled = seg / jnp.maximum(cnt, 1.0)
    h = x1_ref[...] + pooled
    mu = jnp.mean(h, axis=0, keepdims=True)
    d = h - mu
    var = jnp.mean(d * d, axis=0, keepdims=True)
    z = g_ref[...] * d * lax.rsqrt(var + 1e-5) + b_ref[...]
    o_ref[...] = x0_ref[...] + z * _sigmoid(z)


EB = 8000


def _edge_stats_body(w0_ref, g34_ref, wet_ref, be_ref, s_ref, q_ref):
    ep = (
        jnp.dot(w0_ref[...], wet_ref[...], preferred_element_type=jnp.float32)
        + be_ref[...]
        + g34_ref[...]
    )
    bs = jnp.sum(ep, axis=0, keepdims=True)
    bq = jnp.sum(ep * ep, axis=0, keepdims=True)

    @pl.when(pl.program_id(0) == 0)
    def _():
        s_ref[...] = bs
        q_ref[...] = bq

    @pl.when(pl.program_id(0) > 0)
    def _():
        s_ref[...] += bs
        q_ref[...] += bq


def _edge_out_body(w0_ref, g34_ref, wet_ref, be_ref, s_ref, q_ref,
                   g_ref, b_ref, o_ref):
    ep = (
        jnp.dot(w0_ref[...], wet_ref[...], preferred_element_type=jnp.float32)
        + be_ref[...]
        + g34_ref[...]
    )
    inv_e = 1.0 / E
    mu = s_ref[...] * inv_e
    var = q_ref[...] * inv_e - mu * mu
    z = g_ref[...] * (ep - mu) * lax.rsqrt(var + 1e-5) + b_ref[...]
    o_ref[...] = w0_ref[...] + z * _sigmoid(z)


def kernel(x, edge_index, edge_attr, w_v1, b_v1, w_v2, b_v2, w_v3, b_v3,
           w_v4, b_v4, w_e, b_e, bn_v_gamma, bn_v_beta, bn_e_gamma, bn_e_beta):
    src = edge_index[0]
    dst = edge_index[1]

    wt = jnp.concatenate([w_v1.T, w_v2.T, w_v3.T, w_v4.T], axis=1)
    bc = jnp.concatenate([b_v1, b_v2, b_v3, b_v4]).reshape(1, 4 * U)
    x1234 = pl.pallas_call(
        _node_mm_body,
        out_shape=jax.ShapeDtypeStruct((N, 4 * U), jnp.float32),
    )(x, wt, bc)
    x1 = x1234[:, :U]

    def _halves(a):
        return jnp.stack([a[:, :H], a[:, H:]])

    x2s = _halves(x1234[:, U:2 * U])
    x3s = _halves(x1234[:, 2 * U:3 * U])
    x4s = _halves(x1234[:, 3 * U:])

    seg_parts, cnt16, g34 = _sc_call(src, dst, edge_attr, x2s, x3s, x4s)

    x_out = pl.pallas_call(
        _node_out_body,
        out_shape=jax.ShapeDtypeStruct((N, U), jnp.float32),
    )(x, x1, seg_parts, cnt16,
      bn_v_gamma.reshape(1, U), bn_v_beta.reshape(1, U))

    grid = E // EB
    ssum, ssq = pl.pallas_call(
        _edge_stats_body,
        grid=(grid,),
        in_specs=[
            pl.BlockSpec((EB, U), lambda i: (i, 0)),
            pl.BlockSpec((EB, U), lambda i: (i, 0)),
            pl.BlockSpec((U, U), lambda i: (0, 0)),
            pl.BlockSpec((1, U), lambda i: (0, 0)),
        ],
        out_specs=[
            pl.BlockSpec((1, U), lambda i: (0, 0)),
            pl.BlockSpec((1, U), lambda i: (0, 0)),
        ],
        out_shape=[
            jax.ShapeDtypeStruct((1, U), jnp.float32),
            jax.ShapeDtypeStruct((1, U), jnp.float32),
        ],
    )(edge_attr, g34, w_e.T, b_e.reshape(1, U))

    w_out = pl.pallas_call(
        _edge_out_body,
        grid=(grid,),
        in_specs=[
            pl.BlockSpec((EB, U), lambda i: (i, 0)),
            pl.BlockSpec((EB, U), lambda i: (i, 0)),
            pl.BlockSpec((U, U), lambda i: (0, 0)),
            pl.BlockSpec((1, U), lambda i: (0, 0)),
            pl.BlockSpec((1, U), lambda i: (0, 0)),
            pl.BlockSpec((1, U), lambda i: (0, 0)),
            pl.BlockSpec((1, U), lambda i: (0, 0)),
            pl.BlockSpec((1, U), lambda i: (0, 0)),
        ],
        out_specs=pl.BlockSpec((EB, U), lambda i: (i, 0)),
        out_shape=jax.ShapeDtypeStruct((E, U), jnp.float32),
    )(edge_attr, g34, w_e.T, b_e.reshape(1, U), ssum, ssq,
      bn_e_gamma.reshape(1, U), bn_e_beta.reshape(1, U))

    return (x_out, w_out)
